# Initial kernel scaffold; baseline (speedup 1.0000x reference)
import jax
import jax.numpy as jnp
from jax.experimental import pallas as pl


def kernel(query_source, context, cn_gamma, cn_beta, qn_gamma, qn_beta, on_gamma, on_beta, W_kv, W_q, W_out, gamma):
    raise NotImplementedError("write your pallas kernel here")


# speedup vs baseline: 1.5733x; 1.5733x over previous
import functools

import jax
import jax.numpy as jnp
from jax.experimental import pallas as pl
from jax.experimental.pallas import tpu as pltpu

HEADS = 16
DH = 64
INNER = HEADS * DH
TOP_K = 256
KMEANS_ITERS = 5
LT = 256


def _chan_ln_block(x, g, b):
    mean = jnp.mean(x, axis=0, keepdims=True)
    var = jnp.mean((x - mean) * (x - mean), axis=0, keepdims=True)
    return g * (x - mean) / (jnp.sqrt(var) + 1e-6) + b


def _proj_kernel(ctx_ref, qs_ref, wkv_ref, wq_ref, cg_ref, cb_ref, qg_ref,
                 qb_ref, q_ref, k_ref, v_ref):
    ctx_ln = _chan_ln_block(ctx_ref[0], cg_ref[...], cb_ref[...])
    qs_ln = _chan_ln_block(qs_ref[0], qg_ref[...], qb_ref[...])
    kv_t = jax.lax.dot_general(ctx_ln, wkv_ref[...], (((0,), (1,)), ((), ())),
                               preferred_element_type=jnp.float32)
    q_t = jax.lax.dot_general(qs_ln, wq_ref[...], (((0,), (1,)), ((), ())),
                              preferred_element_type=jnp.float32)
    for h in range(HEADS):
        qh = q_t[:, h * DH:(h + 1) * DH]
        qn = jnp.sqrt(jnp.sum(qh * qh, axis=1, keepdims=True))
        q_ref[h] = qh / jnp.maximum(qn, 1e-12)
        kh = kv_t[:, h * DH:(h + 1) * DH]
        kn = jnp.sqrt(jnp.sum(kh * kh, axis=1, keepdims=True))
        k_ref[h] = kh / jnp.maximum(kn, 1e-12)
        v_ref[h] = kv_t[:, INNER + h * DH:INNER + (h + 1) * DH]


def _kmeans_kernel(x_ref, cent_ref):
    n = x_ref.shape[0]
    chunk = 2048
    nchunks = n // chunk

    def one_iter(_, cent):
        cnorm = jnp.sum(cent * cent, axis=1)

        def chunk_body(c, carry):
            sums, counts = carry
            xc = x_ref[pl.ds(c * chunk, chunk), :]
            xnorm = jnp.sum(xc * xc, axis=1, keepdims=True)
            dots = jax.lax.dot_general(xc, cent, (((1,), (1,)), ((), ())),
                                       preferred_element_type=jnp.float32)
            dist = xnorm - 2.0 * dots + cnorm[None, :]
            labels = jnp.argmin(dist, axis=1)
            oh = (labels[:, None] == jax.lax.broadcasted_iota(
                jnp.int32, (chunk, TOP_K), 1)).astype(jnp.float32)
            sums = sums + jax.lax.dot_general(
                oh, xc, (((0,), (0,)), ((), ())),
                preferred_element_type=jnp.float32)
            counts = counts + jax.lax.dot_general(
                oh, jnp.ones((chunk, 8), jnp.float32), (((0,), (0,)), ((), ())),
                preferred_element_type=jnp.float32)
            return sums, counts

        sums, counts = jax.lax.fori_loop(
            0, nchunks, chunk_body,
            (jnp.zeros((TOP_K, DH), jnp.float32),
             jnp.zeros((TOP_K, 8), jnp.float32)))
        cnt = counts[:, :1]
        return jnp.where(cnt > 0, sums / jnp.maximum(cnt, 1.0), cent)

    cent_ref[...] = jax.lax.fori_loop(0, KMEANS_ITERS, one_iter,
                                      x_ref[0:TOP_K, :])


def _cumsum_lanes(a):
    rows, cols = a.shape
    lane = jax.lax.broadcasted_iota(jnp.int32, (rows, cols), 1)
    s = 1
    while s < cols:
        sh = pltpu.roll(a, s, 1)
        a = a + jnp.where(lane >= s, sh, 0)
        s *= 2
    return a


def _cumsum_sublanes(a):
    rows, cols = a.shape
    sub = jax.lax.broadcasted_iota(jnp.int32, (rows, cols), 0)
    s = 1
    while s < rows:
        sh = pltpu.roll(a, s, 0)
        a = a + jnp.where(sub >= s, sh, 0)
        s *= 2
    return a


def _cumsum2048(flat):
    a = flat
    within = _cumsum_lanes(a)
    row_tot = jnp.sum(a, axis=1, keepdims=True)
    row_off = _cumsum_sublanes(row_tot) - row_tot
    return within + row_off


def _select_kernel(k_ref, v_ref, cent_ref, ksel_ref, vsel_ref):
    k = k_ref[0]
    v = v_ref[0]
    cent = cent_ref[...]
    L = k.shape[0]
    cnorm = jnp.sum(cent * cent, axis=1)
    knorm = jnp.sum(k * k, axis=1, keepdims=True)
    dots = jax.lax.dot_general(k, cent, (((1,), (1,)), ((), ())),
                               preferred_element_type=jnp.float32)
    dist = knorm - 2.0 * dots + cnorm[None, :]
    labels = jnp.argmin(dist, axis=1)
    oh = (labels[:, None] == jax.lax.broadcasted_iota(
        jnp.int32, (L, TOP_K), 1)).astype(jnp.float32)
    kc = jax.lax.dot_general(oh, cent, (((1,), (0,)), ((), ())),
                             preferred_element_type=jnp.float32)
    kd = jnp.sum(jnp.abs(kc - k), axis=1)

    bits = jax.lax.bitcast_convert_type(kd, jnp.int32).reshape(16, L // 16)

    def bs_body(_, lohi):
        lo, hi = lohi
        mid = lo + (hi - lo + 1) // 2
        cnt = jnp.sum(jnp.where(bits >= mid, 1, 0))
        big = cnt >= TOP_K
        return jnp.where(big, mid, lo), jnp.where(big, hi, mid - 1)

    lo0 = jnp.int32(0)
    hi0 = jnp.max(bits)
    t, _ = jax.lax.fori_loop(0, 31, bs_body, (lo0, hi0))

    mask_gt = bits > t
    n_gt = jnp.sum(jnp.where(mask_gt, 1, 0))
    mask_eq = bits == t
    eq_cum = _cumsum2048(jnp.where(mask_eq, 1, 0))
    sel = mask_gt | (mask_eq & (eq_cum <= TOP_K - n_gt))
    sel_i = jnp.where(sel, 1, 0)
    rank_excl = (_cumsum2048(sel_i) - sel_i).reshape(1, L)
    sel_row = sel.reshape(1, L)
    slot = jax.lax.broadcasted_iota(jnp.int32, (TOP_K, L), 0)
    scatter = ((slot == rank_excl) & sel_row).astype(jnp.float32)
    ksel_ref[0] = jax.lax.dot_general(scatter, k, (((1,), (0,)), ((), ())),
                                      preferred_element_type=jnp.float32)
    vsel_ref[0] = jax.lax.dot_general(scatter, v, (((1,), (0,)), ((), ())),
                                      preferred_element_type=jnp.float32)


def _attn_kernel(q_ref, ksel_ref, vsel_ref, out_ref):
    q = q_ref[0]
    ks = ksel_ref[0]
    vs = vsel_ref[0]
    sim = jax.lax.dot_general(q, ks, (((1,), (1,)), ((), ())),
                              preferred_element_type=jnp.float32)
    m = jnp.max(sim, axis=1, keepdims=True)
    e = jnp.exp(sim - m)
    a = e / jnp.sum(e, axis=1, keepdims=True)
    out_ref[0] = jax.lax.dot_general(vs, a, (((0,), (1,)), ((), ())),
                                     preferred_element_type=jnp.float32)


def _outproj_kernel(a_ref, qs_ref, wout_ref, og_ref, ob_ref, gamma_ref,
                    out_ref):
    res = jnp.zeros((INNER, LT), jnp.float32)
    for h in range(HEADS):
        res = res + jax.lax.dot_general(
            wout_ref[:, h * DH:(h + 1) * DH], a_ref[h],
            (((1,), (0,)), ((), ())), preferred_element_type=jnp.float32)
    ln = _chan_ln_block(res, og_ref[...], ob_ref[...])
    out_ref[0] = gamma_ref[0, 0] * ln + qs_ref[0]


def kernel(query_source, context, cn_gamma, cn_beta, qn_gamma, qn_beta,
           on_gamma, on_beta, W_kv, W_q, W_out, gamma):
    b, dim, Lq = query_source.shape
    Lc = context.shape[2]
    BH = b * HEADS
    f32 = jnp.float32

    cg = cn_gamma.reshape(dim, 1)
    cb = cn_beta.reshape(dim, 1)
    qg = qn_gamma.reshape(dim, 1)
    qb = qn_beta.reshape(dim, 1)
    og = on_gamma.reshape(dim, 1)
    ob = on_beta.reshape(dim, 1)
    gm = gamma.reshape(1, 1)

    n_lt = Lq // LT
    q_t, k_t, v_t = pl.pallas_call(
        _proj_kernel,
        grid=(b, n_lt),
        in_specs=[
            pl.BlockSpec((1, dim, LT), lambda i, j: (i, 0, j)),
            pl.BlockSpec((1, dim, LT), lambda i, j: (i, 0, j)),
            pl.BlockSpec((2 * INNER, dim), lambda i, j: (0, 0)),
            pl.BlockSpec((INNER, dim), lambda i, j: (0, 0)),
            pl.BlockSpec((dim, 1), lambda i, j: (0, 0)),
            pl.BlockSpec((dim, 1), lambda i, j: (0, 0)),
            pl.BlockSpec((dim, 1), lambda i, j: (0, 0)),
            pl.BlockSpec((dim, 1), lambda i, j: (0, 0)),
        ],
        out_specs=[
            pl.BlockSpec((HEADS, LT, DH), lambda i, j: (i, j, 0)),
            pl.BlockSpec((HEADS, LT, DH), lambda i, j: (i, j, 0)),
            pl.BlockSpec((HEADS, LT, DH), lambda i, j: (i, j, 0)),
        ],
        out_shape=[
            jax.ShapeDtypeStruct((BH, Lq, DH), f32),
            jax.ShapeDtypeStruct((BH, Lc, DH), f32),
            jax.ShapeDtypeStruct((BH, Lc, DH), f32),
        ],
    )(context, query_source, W_kv, W_q, cg, cb, qg, qb)

    x = q_t.reshape(BH * Lq, DH)
    centroids = pl.pallas_call(
        _kmeans_kernel,
        out_shape=jax.ShapeDtypeStruct((TOP_K, DH), f32),
    )(x)

    k_sel, v_sel = pl.pallas_call(
        _select_kernel,
        grid=(BH,),
        in_specs=[
            pl.BlockSpec((1, Lc, DH), lambda i: (i, 0, 0)),
            pl.BlockSpec((1, Lc, DH), lambda i: (i, 0, 0)),
            pl.BlockSpec((TOP_K, DH), lambda i: (0, 0)),
        ],
        out_specs=[
            pl.BlockSpec((1, TOP_K, DH), lambda i: (i, 0, 0)),
            pl.BlockSpec((1, TOP_K, DH), lambda i: (i, 0, 0)),
        ],
        out_shape=[
            jax.ShapeDtypeStruct((BH, TOP_K, DH), f32),
            jax.ShapeDtypeStruct((BH, TOP_K, DH), f32),
        ],
    )(k_t, v_t, centroids)

    LQT = 512
    attn_out = pl.pallas_call(
        _attn_kernel,
        grid=(BH, Lq // LQT),
        in_specs=[
            pl.BlockSpec((1, LQT, DH), lambda i, j: (i, j, 0)),
            pl.BlockSpec((1, TOP_K, DH), lambda i, j: (i, 0, 0)),
            pl.BlockSpec((1, TOP_K, DH), lambda i, j: (i, 0, 0)),
        ],
        out_specs=pl.BlockSpec((1, DH, LQT), lambda i, j: (i, 0, j)),
        out_shape=jax.ShapeDtypeStruct((BH, DH, Lq), f32),
    )(q_t, k_sel, v_sel)

    out = pl.pallas_call(
        _outproj_kernel,
        grid=(b, n_lt),
        in_specs=[
            pl.BlockSpec((HEADS, DH, LT), lambda i, j: (i, 0, j)),
            pl.BlockSpec((1, dim, LT), lambda i, j: (i, 0, j)),
            pl.BlockSpec((dim, INNER), lambda i, j: (0, 0)),
            pl.BlockSpec((dim, 1), lambda i, j: (0, 0)),
            pl.BlockSpec((dim, 1), lambda i, j: (0, 0)),
            pl.BlockSpec((1, 1), lambda i, j: (0, 0)),
        ],
        out_specs=pl.BlockSpec((1, dim, LT), lambda i, j: (i, 0, j)),
        out_shape=jax.ShapeDtypeStruct((b, dim, Lq), f32),
    )(attn_out, query_source, W_out, og, ob, gm)

    return out
